# baseline (speedup 1.0000x reference)
import functools

import jax
import jax.numpy as jnp
from jax import lax
from jax.experimental import pallas as pl
from jax.experimental.pallas import tpu as pltpu
from jax.experimental.pallas import tpu_sc as plsc

N_NODES = 10000
D = 128
N_CLASSES = 10
BN_EPS = 1e-5

NC = 2
NS = 16
NW = NC * NS

N_PAD = 10240
RPT = N_PAD // NS
CH = 48
NB = 4
SB = 56
NBLK = 4


def _seg_body(h_hbm, src_hbm, dst_hbm, zeros_hbm, out_hbm,
              sidx, didx, r0, r1, r2, r3, acc,
              g0, g1, g2, g3, s0, s1, s2, s3):
    rows = [r0, r1, r2, r3]
    sg = [g0, g1, g2, g3]
    ss = [s0, s1, s2, s3]
    cid = lax.axis_index("c")
    sid = lax.axis_index("s")
    wid = cid * NS + sid
    pltpu.sync_copy(zeros_hbm, acc.at[pl.ds(sid * RPT, RPT)])
    plsc.subcore_barrier()

    def block_body(k, carry0):
        pltpu.sync_copy(src_hbm.at[wid, k], sidx)
        pltpu.sync_copy(dst_hbm.at[wid, k], didx)
        for b in range(2):
            pltpu.async_copy(h_hbm.at[sidx.at[b]], rows[b], sg[b])

        def round_body(t, carry):
            for b in range(NB):
                g = t * NB + b
                pltpu.make_async_copy(h_hbm.at[sidx.at[g]], rows[b],
                                      sg[b]).wait()
                pltpu.async_copy(rows[b], acc.at[didx.at[g]], ss[b], add=True)
                bp = (b + 2) % NB

                @pl.when(g >= 2)
                def _():
                    pltpu.make_async_copy(rows[bp], acc.at[didx.at[g - 2]],
                                          ss[bp]).wait()

                @pl.when(g + 2 < SB)
                def _():
                    pltpu.async_copy(h_hbm.at[sidx.at[g + 2]], rows[bp],
                                     sg[bp])
            return carry

        lax.fori_loop(0, SB // NB, round_body, 0)
        for g in (SB - 2, SB - 1):
            pltpu.make_async_copy(rows[g % NB], acc.at[didx.at[g]],
                                  ss[g % NB]).wait()
        return carry0

    lax.fori_loop(0, NBLK, block_body, 0)
    plsc.subcore_barrier()
    pltpu.sync_copy(acc.at[pl.ds(sid * RPT, RPT)],
                    out_hbm.at[pl.ds(cid * N_PAD + sid * RPT, RPT)])


def _segment_partials(h, src_p, dst_p, zeros):
    gch = NBLK * SB
    mesh = plsc.VectorSubcoreMesh(core_axis_name="c", subcore_axis_name="s")
    kfn = pl.kernel(
        _seg_body,
        out_type=jax.ShapeDtypeStruct((NC * N_PAD, D), jnp.float32),
        mesh=mesh,
        scratch_types=[
            pltpu.VMEM((SB, CH), jnp.int32),
            pltpu.VMEM((SB, CH), jnp.int32),
        ] + [pltpu.VMEM((CH, D), jnp.float32)] * NB + [
            pltpu.VMEM_SHARED((N_PAD, D), jnp.float32),
        ] + [pltpu.SemaphoreType.DMA] * (2 * NB),
    )
    del gch
    return kfn(h, src_p.reshape(NW, NBLK, SB, CH),
               dst_p.reshape(NW, NBLK, SB, CH), zeros)


def _linear_body(x_ref, w_ref, b_ref, o_ref):
    o_ref[...] = jnp.dot(x_ref[...], w_ref[...],
                         preferred_element_type=jnp.float32) + b_ref[...]


def _linear(x, w, b):
    n = x.shape[0]
    return pl.pallas_call(
        _linear_body,
        out_shape=jax.ShapeDtypeStruct((n, w.shape[1]), jnp.float32),
    )(x, w, b.reshape(1, -1))


def _bn(h, g, e):
    m = jnp.mean(h, axis=0, keepdims=True)
    v = jnp.mean(jnp.square(h - m), axis=0, keepdims=True)
    return (h - m) * (g * lax.rsqrt(v + BN_EPS)) + e


def _mlp_body(h_ref, p0_ref, p1_ref, w1_ref, b1_ref, g1_ref, e1_ref,
              w2_ref, b2_ref, g2_ref, e2_ref, o_ref):
    z = h_ref[...] + p0_ref[...] + p1_ref[...]
    h1 = jnp.dot(z, w1_ref[...], preferred_element_type=jnp.float32) + b1_ref[...]
    h1 = jnp.maximum(_bn(h1, g1_ref[...], e1_ref[...]), 0.0)
    h2 = jnp.dot(h1, w2_ref[...], preferred_element_type=jnp.float32) + b2_ref[...]
    o_ref[...] = jnp.maximum(_bn(h2, g2_ref[...], e2_ref[...]), 0.0)


def _mlp(h, p0, p1, p):
    n = h.shape[0]
    r = lambda a: a.reshape(1, -1)
    return pl.pallas_call(
        _mlp_body,
        out_shape=jax.ShapeDtypeStruct((n, p['W2'].shape[1]), jnp.float32),
    )(h, p0, p1, p['W1'], r(p['b1']), r(p['g1']), r(p['be1']),
      p['W2'], r(p['b2']), r(p['g2']), r(p['be2']))


def _final_body(h_ref, m_ref, w_ref, b_ref, o_ref):
    z = h_ref[...] * m_ref[...]
    o_ref[...] = jnp.dot(z, w_ref[...],
                         preferred_element_type=jnp.float32) + b_ref[...]


def _final(h, maskf, w, b):
    n = h.shape[0]
    return pl.pallas_call(
        _final_body,
        out_shape=jax.ShapeDtypeStruct((n, w.shape[1]), jnp.float32),
    )(h, maskf, w, b.reshape(1, -1))


def kernel(x, edge_index, mask, params):
    n = x.shape[0]
    e = edge_index.shape[1]
    src = edge_index[0].astype(jnp.int32)
    dst = edge_index[1].astype(jnp.int32)
    epw = NW * CH * SB * NBLK
    e_pad = ((e + epw - 1) // epw) * epw
    pad = e_pad - e
    if pad:
        src = jnp.concatenate([src, jnp.zeros((pad,), jnp.int32)])
        dst = jnp.concatenate([dst, jnp.full((pad,), N_PAD - 8, jnp.int32)])
    zeros = jnp.zeros((RPT, D), jnp.float32)

    h = _linear(x, params['init_W'], params['init_b'])
    for p in params['convs']:
        parts = _segment_partials(h, src, dst, zeros)
        h = _mlp(h, parts[0:n], parts[N_PAD:N_PAD + n], p)

    maskf = mask.astype(jnp.float32)[:, None]
    wp = jnp.pad(params['lin_W'], ((0, 0), (0, D - N_CLASSES)))
    bp = jnp.pad(params['lin_b'], (0, D - N_CLASSES))
    out = _final(h, maskf, wp, bp)
    return out[:, :N_CLASSES]


# speedup vs baseline: 2.3022x; 2.3022x over previous
import functools

import jax
import jax.numpy as jnp
from jax import lax
from jax.experimental import pallas as pl
from jax.experimental.pallas import tpu as pltpu
from jax.experimental.pallas import tpu_sc as plsc

N_NODES = 10000
D = 128
N_CLASSES = 10
BN_EPS = 1e-5

NC = 2
NS = 16
NW = NC * NS

N_PAD = 10240
RPT = N_PAD // NS
CH = 128
SB = 40
NBLK = 2


def _seg_body(h_hbm, eidx_hbm, zeros_hbm, out_hbm, idx, rows, acc, sem):
    cid = lax.axis_index("c")
    sid = lax.axis_index("s")
    wid = cid * NS + sid
    pltpu.sync_copy(zeros_hbm, acc.at[pl.ds(sid * RPT, RPT)])
    plsc.subcore_barrier()

    def block_body(k, carry0):
        pltpu.sync_copy(eidx_hbm.at[wid, k], idx)

        def chunk_body(g, carry):
            pltpu.async_copy(h_hbm.at[idx.at[0, g]], rows, sem).wait()
            pltpu.sync_copy(rows, acc.at[idx.at[1, g]], add=True)
            return carry

        lax.fori_loop(0, SB, chunk_body, 0)
        return carry0

    lax.fori_loop(0, NBLK, block_body, 0)
    plsc.subcore_barrier()
    pltpu.sync_copy(acc.at[pl.ds(sid * RPT, RPT)],
                    out_hbm.at[pl.ds(cid * N_PAD + sid * RPT, RPT)])


def _segment_partials(h, src_p, dst_p, zeros):
    gch = NBLK * SB
    mesh = plsc.VectorSubcoreMesh(core_axis_name="c", subcore_axis_name="s")
    kfn = pl.kernel(
        _seg_body,
        out_type=jax.ShapeDtypeStruct((NC * N_PAD, D), jnp.float32),
        mesh=mesh,
        scratch_types=[
            pltpu.VMEM((2, SB, CH), jnp.int32),
            pltpu.VMEM((CH, D), jnp.float32),
            pltpu.VMEM_SHARED((N_PAD, D), jnp.float32),
            pltpu.SemaphoreType.DMA,
        ],
    )
    eidx = jnp.stack([src_p.reshape(NW, NBLK, SB, CH),
                      dst_p.reshape(NW, NBLK, SB, CH)], axis=2)
    return kfn(h, eidx, zeros)


def _linear_body(x_ref, w_ref, b_ref, o_ref):
    o_ref[...] = jnp.dot(x_ref[...], w_ref[...],
                         preferred_element_type=jnp.float32) + b_ref[...]


def _linear(x, w, b):
    n = x.shape[0]
    return pl.pallas_call(
        _linear_body,
        out_shape=jax.ShapeDtypeStruct((n, w.shape[1]), jnp.float32),
    )(x, w, b.reshape(1, -1))


def _bn(h, g, e):
    m = jnp.mean(h, axis=0, keepdims=True)
    v = jnp.mean(jnp.square(h - m), axis=0, keepdims=True)
    return (h - m) * (g * lax.rsqrt(v + BN_EPS)) + e


def _mlp_body(h_ref, p0_ref, p1_ref, w1_ref, b1_ref, g1_ref, e1_ref,
              w2_ref, b2_ref, g2_ref, e2_ref, o_ref):
    z = h_ref[...] + p0_ref[...] + p1_ref[...]
    h1 = jnp.dot(z, w1_ref[...], preferred_element_type=jnp.float32) + b1_ref[...]
    h1 = jnp.maximum(_bn(h1, g1_ref[...], e1_ref[...]), 0.0)
    h2 = jnp.dot(h1, w2_ref[...], preferred_element_type=jnp.float32) + b2_ref[...]
    o_ref[...] = jnp.maximum(_bn(h2, g2_ref[...], e2_ref[...]), 0.0)


def _mlp(h, p0, p1, p):
    n = h.shape[0]
    r = lambda a: a.reshape(1, -1)
    return pl.pallas_call(
        _mlp_body,
        out_shape=jax.ShapeDtypeStruct((n, p['W2'].shape[1]), jnp.float32),
    )(h, p0, p1, p['W1'], r(p['b1']), r(p['g1']), r(p['be1']),
      p['W2'], r(p['b2']), r(p['g2']), r(p['be2']))


def _final_body(h_ref, m_ref, w_ref, b_ref, o_ref):
    z = h_ref[...] * m_ref[...]
    o_ref[...] = jnp.dot(z, w_ref[...],
                         preferred_element_type=jnp.float32) + b_ref[...]


def _final(h, maskf, w, b):
    n = h.shape[0]
    return pl.pallas_call(
        _final_body,
        out_shape=jax.ShapeDtypeStruct((n, w.shape[1]), jnp.float32),
    )(h, maskf, w, b.reshape(1, -1))


def kernel(x, edge_index, mask, params):
    n = x.shape[0]
    e = edge_index.shape[1]
    src = edge_index[0].astype(jnp.int32)
    dst = edge_index[1].astype(jnp.int32)
    epw = NW * CH * SB * NBLK
    e_pad = ((e + epw - 1) // epw) * epw
    pad = e_pad - e
    if pad:
        src = jnp.concatenate([src, jnp.zeros((pad,), jnp.int32)])
        dst = jnp.concatenate([dst, jnp.full((pad,), N_PAD - 8, jnp.int32)])
    zeros = jnp.zeros((RPT, D), jnp.float32)

    h = _linear(x, params['init_W'], params['init_b'])
    for p in params['convs']:
        parts = _segment_partials(h, src, dst, zeros)
        h = _mlp(h, parts[0:n], parts[N_PAD:N_PAD + n], p)

    maskf = mask.astype(jnp.float32)[:, None]
    wp = jnp.pad(params['lin_W'], ((0, 0), (0, D - N_CLASSES)))
    bp = jnp.pad(params['lin_b'], (0, D - N_CLASSES))
    out = _final(h, maskf, wp, bp)
    return out[:, :N_CLASSES]


# speedup vs baseline: 3.2500x; 1.4117x over previous
import functools

import jax
import jax.numpy as jnp
from jax import lax
from jax.experimental import pallas as pl
from jax.experimental.pallas import tpu as pltpu
from jax.experimental.pallas import tpu_sc as plsc

N_NODES = 10000
D = 128
N_CLASSES = 10
BN_EPS = 1e-5

NC = 2
NS = 16
NW = NC * NS

N_PAD = 10240
RPT = N_PAD // NS
CH = 128


def _seg_body(h_hbm, src_hbm, dst_hbm, zeros_hbm, out_hbm,
              sidx, didx, rows, acc, sem, *, ept):
    cid = lax.axis_index("c")
    sid = lax.axis_index("s")
    wid = cid * NS + sid
    pltpu.sync_copy(zeros_hbm, acc.at[pl.ds(sid * RPT, RPT)])
    pltpu.sync_copy(src_hbm.at[pl.ds(wid * ept, ept)], sidx)
    plsc.subcore_barrier()
    ebase = wid * ept

    def chunk_body(g, carry):
        pltpu.async_copy(h_hbm.at[sidx.at[pl.ds(g * CH, CH)]], rows,
                         sem).wait()
        pltpu.sync_copy(dst_hbm.at[pl.ds(ebase + g * CH, CH)], didx)
        pltpu.sync_copy(rows, acc.at[didx], add=True)
        return carry

    lax.fori_loop(0, ept // CH, chunk_body, 0)
    plsc.subcore_barrier()
    pltpu.sync_copy(acc.at[pl.ds(sid * RPT, RPT)],
                    out_hbm.at[pl.ds(cid * N_PAD + sid * RPT, RPT)])


def _segment_partials(h, src_p, dst_p, zeros):
    ept = src_p.shape[0] // NW
    mesh = plsc.VectorSubcoreMesh(core_axis_name="c", subcore_axis_name="s")
    kfn = pl.kernel(
        functools.partial(_seg_body, ept=ept),
        out_type=jax.ShapeDtypeStruct((NC * N_PAD, D), jnp.float32),
        mesh=mesh,
        scratch_types=[
            pltpu.VMEM((ept,), jnp.int32),
            pltpu.VMEM((CH,), jnp.int32),
            pltpu.VMEM((CH, D), jnp.float32),
            pltpu.VMEM_SHARED((N_PAD, D), jnp.float32),
            pltpu.SemaphoreType.DMA,
        ],
    )
    return kfn(h, src_p, dst_p, zeros)


def _linear_body(x_ref, w_ref, b_ref, o_ref):
    o_ref[...] = jnp.dot(x_ref[...], w_ref[...],
                         preferred_element_type=jnp.float32) + b_ref[...]


def _linear(x, w, b):
    n = x.shape[0]
    return pl.pallas_call(
        _linear_body,
        out_shape=jax.ShapeDtypeStruct((n, w.shape[1]), jnp.float32),
    )(x, w, b.reshape(1, -1))


def _bn(h, g, e):
    m = jnp.mean(h, axis=0, keepdims=True)
    v = jnp.mean(jnp.square(h - m), axis=0, keepdims=True)
    return (h - m) * (g * lax.rsqrt(v + BN_EPS)) + e


def _mlp_body(h_ref, p0_ref, p1_ref, w1_ref, b1_ref, g1_ref, e1_ref,
              w2_ref, b2_ref, g2_ref, e2_ref, o_ref):
    z = h_ref[...] + p0_ref[...] + p1_ref[...]
    h1 = jnp.dot(z, w1_ref[...], preferred_element_type=jnp.float32) + b1_ref[...]
    h1 = jnp.maximum(_bn(h1, g1_ref[...], e1_ref[...]), 0.0)
    h2 = jnp.dot(h1, w2_ref[...], preferred_element_type=jnp.float32) + b2_ref[...]
    o_ref[...] = jnp.maximum(_bn(h2, g2_ref[...], e2_ref[...]), 0.0)


def _mlp(h, p0, p1, p):
    n = h.shape[0]
    r = lambda a: a.reshape(1, -1)
    return pl.pallas_call(
        _mlp_body,
        out_shape=jax.ShapeDtypeStruct((n, p['W2'].shape[1]), jnp.float32),
    )(h, p0, p1, p['W1'], r(p['b1']), r(p['g1']), r(p['be1']),
      p['W2'], r(p['b2']), r(p['g2']), r(p['be2']))


def _final_body(h_ref, m_ref, w_ref, b_ref, o_ref):
    z = h_ref[...] * m_ref[...]
    o_ref[...] = jnp.dot(z, w_ref[...],
                         preferred_element_type=jnp.float32) + b_ref[...]


def _final(h, maskf, w, b):
    n = h.shape[0]
    return pl.pallas_call(
        _final_body,
        out_shape=jax.ShapeDtypeStruct((n, w.shape[1]), jnp.float32),
    )(h, maskf, w, b.reshape(1, -1))


def kernel(x, edge_index, mask, params):
    n = x.shape[0]
    e = edge_index.shape[1]
    src = edge_index[0].astype(jnp.int32)
    dst = edge_index[1].astype(jnp.int32)
    epw = NW * CH
    e_pad = ((e + epw - 1) // epw) * epw
    pad = e_pad - e
    if pad:
        src = jnp.concatenate([src, jnp.zeros((pad,), jnp.int32)])
        dst = jnp.concatenate([dst, jnp.full((pad,), N_PAD - 8, jnp.int32)])
    zeros = jnp.zeros((RPT, D), jnp.float32)

    h = _linear(x, params['init_W'], params['init_b'])
    for p in params['convs']:
        parts = _segment_partials(h, src, dst, zeros)
        h = _mlp(h, parts[0:n], parts[N_PAD:N_PAD + n], p)

    maskf = mask.astype(jnp.float32)[:, None]
    wp = jnp.pad(params['lin_W'], ((0, 0), (0, D - N_CLASSES)))
    bp = jnp.pad(params['lin_b'], (0, D - N_CLASSES))
    out = _final(h, maskf, wp, bp)
    return out[:, :N_CLASSES]


# speedup vs baseline: 4.5601x; 1.4031x over previous
import functools

import jax
import jax.numpy as jnp
from jax import lax
from jax.experimental import pallas as pl
from jax.experimental.pallas import tpu as pltpu
from jax.experimental.pallas import tpu_sc as plsc

N_NODES = 10000
D = 128
N_CLASSES = 10
BN_EPS = 1e-5

NC = 2
NS = 16
NW = NC * NS

N_PAD = 10240
RPT = N_PAD // NS
CH = 120


def _seg_body(h_hbm, src_hbm, dst_hbm, zeros_hbm, out_hbm,
              sa, sb, didx, ra, rb, acc, ga, gb, *, ept):
    sidx = [sa, sb]
    rows = [ra, rb]
    sems = [ga, gb]
    cid = lax.axis_index("c")
    sid = lax.axis_index("s")
    wid = cid * NS + sid
    gch = ept // CH
    pltpu.sync_copy(zeros_hbm, acc.at[pl.ds(sid * RPT, RPT)])
    plsc.subcore_barrier()
    ebase = wid * ept

    pltpu.sync_copy(src_hbm.at[pl.ds(ebase, CH)], sa)
    pltpu.async_copy(h_hbm.at[sa], ra, ga)

    def round_body(t, carry):
        for b in range(2):
            g = t * 2 + b
            cur, nxt = b, 1 - b

            @pl.when(g + 1 < gch)
            def _():
                pltpu.sync_copy(src_hbm.at[pl.ds(ebase + (g + 1) * CH, CH)],
                                sidx[nxt])
                pltpu.async_copy(h_hbm.at[sidx[nxt]], rows[nxt], sems[nxt])

            pltpu.make_async_copy(h_hbm.at[sidx[cur]], rows[cur],
                                  sems[cur]).wait()
            pltpu.sync_copy(dst_hbm.at[pl.ds(ebase + g * CH, CH)], didx)
            pltpu.sync_copy(rows[cur], acc.at[didx], add=True)
        return carry

    lax.fori_loop(0, gch // 2, round_body, 0)
    plsc.subcore_barrier()
    pltpu.sync_copy(acc.at[pl.ds(sid * RPT, RPT)],
                    out_hbm.at[pl.ds(cid * N_PAD + sid * RPT, RPT)])


def _segment_partials(h, src_p, dst_p, zeros):
    ept = src_p.shape[0] // NW
    mesh = plsc.VectorSubcoreMesh(core_axis_name="c", subcore_axis_name="s")
    kfn = pl.kernel(
        functools.partial(_seg_body, ept=ept),
        out_type=jax.ShapeDtypeStruct((NC * N_PAD, D), jnp.float32),
        mesh=mesh,
        scratch_types=[
            pltpu.VMEM((CH,), jnp.int32),
            pltpu.VMEM((CH,), jnp.int32),
            pltpu.VMEM((CH,), jnp.int32),
            pltpu.VMEM((CH, D), jnp.float32),
            pltpu.VMEM((CH, D), jnp.float32),
            pltpu.VMEM_SHARED((N_PAD, D), jnp.float32),
            pltpu.SemaphoreType.DMA,
            pltpu.SemaphoreType.DMA,
        ],
    )
    return kfn(h, src_p, dst_p, zeros)


def _linear_body(x_ref, w_ref, b_ref, o_ref):
    o_ref[...] = jnp.dot(x_ref[...], w_ref[...],
                         preferred_element_type=jnp.float32) + b_ref[...]


def _linear(x, w, b):
    n = x.shape[0]
    return pl.pallas_call(
        _linear_body,
        out_shape=jax.ShapeDtypeStruct((n, w.shape[1]), jnp.float32),
    )(x, w, b.reshape(1, -1))


def _bn(h, g, e):
    m = jnp.mean(h, axis=0, keepdims=True)
    v = jnp.mean(jnp.square(h - m), axis=0, keepdims=True)
    return (h - m) * (g * lax.rsqrt(v + BN_EPS)) + e


def _mlp_body(h_ref, p0_ref, p1_ref, w1_ref, b1_ref, g1_ref, e1_ref,
              w2_ref, b2_ref, g2_ref, e2_ref, o_ref):
    z = h_ref[...] + p0_ref[...] + p1_ref[...]
    h1 = jnp.dot(z, w1_ref[...], preferred_element_type=jnp.float32) + b1_ref[...]
    h1 = jnp.maximum(_bn(h1, g1_ref[...], e1_ref[...]), 0.0)
    h2 = jnp.dot(h1, w2_ref[...], preferred_element_type=jnp.float32) + b2_ref[...]
    o_ref[...] = jnp.maximum(_bn(h2, g2_ref[...], e2_ref[...]), 0.0)


def _mlp(h, p0, p1, p):
    n = h.shape[0]
    r = lambda a: a.reshape(1, -1)
    return pl.pallas_call(
        _mlp_body,
        out_shape=jax.ShapeDtypeStruct((n, p['W2'].shape[1]), jnp.float32),
    )(h, p0, p1, p['W1'], r(p['b1']), r(p['g1']), r(p['be1']),
      p['W2'], r(p['b2']), r(p['g2']), r(p['be2']))


def _final_body(h_ref, m_ref, w_ref, b_ref, o_ref):
    z = h_ref[...] * m_ref[...]
    o_ref[...] = jnp.dot(z, w_ref[...],
                         preferred_element_type=jnp.float32) + b_ref[...]


def _final(h, maskf, w, b):
    n = h.shape[0]
    return pl.pallas_call(
        _final_body,
        out_shape=jax.ShapeDtypeStruct((n, w.shape[1]), jnp.float32),
    )(h, maskf, w, b.reshape(1, -1))


def kernel(x, edge_index, mask, params):
    n = x.shape[0]
    e = edge_index.shape[1]
    src = edge_index[0].astype(jnp.int32)
    dst = edge_index[1].astype(jnp.int32)
    epw = NW * CH * 2
    e_pad = ((e + epw - 1) // epw) * epw
    pad = e_pad - e
    if pad:
        src = jnp.concatenate([src, jnp.zeros((pad,), jnp.int32)])
        dst = jnp.concatenate([dst, jnp.full((pad,), N_PAD - 8, jnp.int32)])
    zeros = jnp.zeros((RPT, D), jnp.float32)

    h = _linear(x, params['init_W'], params['init_b'])
    for p in params['convs']:
        parts = _segment_partials(h, src, dst, zeros)
        h = _mlp(h, parts[0:n], parts[N_PAD:N_PAD + n], p)

    maskf = mask.astype(jnp.float32)[:, None]
    wp = jnp.pad(params['lin_W'], ((0, 0), (0, D - N_CLASSES)))
    bp = jnp.pad(params['lin_b'], (0, D - N_CLASSES))
    out = _final(h, maskf, wp, bp)
    return out[:, :N_CLASSES]


# speedup vs baseline: 4.7594x; 1.0437x over previous
import functools

import jax
import jax.numpy as jnp
from jax import lax
from jax.experimental import pallas as pl
from jax.experimental.pallas import tpu as pltpu
from jax.experimental.pallas import tpu_sc as plsc

N_NODES = 10000
D = 128
N_CLASSES = 10
BN_EPS = 1e-5

NC = 2
NS = 16
NW = NC * NS

N_PAD = 10240
RPT = N_PAD // NS
CH = 120


def _seg_body(h_hbm, eidx_hbm, zeros_hbm, out_hbm,
              ia, ib, ra, rb, acc, ga, gb, *, gch):
    idx = [ia, ib]
    rows = [ra, rb]
    sems = [ga, gb]
    cid = lax.axis_index("c")
    sid = lax.axis_index("s")
    wid = cid * NS + sid
    pltpu.sync_copy(zeros_hbm, acc.at[pl.ds(sid * RPT, RPT)])
    plsc.subcore_barrier()
    ibase = wid * gch

    pltpu.sync_copy(eidx_hbm.at[ibase], ia)
    pltpu.async_copy(h_hbm.at[ia.at[0]], ra, ga)

    def round_body(t, carry):
        for b in range(2):
            g = t * 2 + b
            cur, nxt = b, 1 - b

            @pl.when(g + 1 < gch)
            def _():
                pltpu.sync_copy(eidx_hbm.at[ibase + g + 1], idx[nxt])
                pltpu.async_copy(h_hbm.at[idx[nxt].at[0]], rows[nxt],
                                 sems[nxt])

            pltpu.make_async_copy(h_hbm.at[idx[cur].at[0]], rows[cur],
                                  sems[cur]).wait()
            pltpu.sync_copy(rows[cur], acc.at[idx[cur].at[1]], add=True)
        return carry

    lax.fori_loop(0, gch // 2, round_body, 0)
    plsc.subcore_barrier()
    pltpu.sync_copy(acc.at[pl.ds(sid * RPT, RPT)],
                    out_hbm.at[pl.ds(cid * N_PAD + sid * RPT, RPT)])


def _segment_partials(h, src_p, dst_p, zeros):
    ept = src_p.shape[0] // NW
    gch = ept // CH
    mesh = plsc.VectorSubcoreMesh(core_axis_name="c", subcore_axis_name="s")
    kfn = pl.kernel(
        functools.partial(_seg_body, gch=gch),
        out_type=jax.ShapeDtypeStruct((NC * N_PAD, D), jnp.float32),
        mesh=mesh,
        scratch_types=[
            pltpu.VMEM((2, CH), jnp.int32),
            pltpu.VMEM((2, CH), jnp.int32),
            pltpu.VMEM((CH, D), jnp.float32),
            pltpu.VMEM((CH, D), jnp.float32),
            pltpu.VMEM_SHARED((N_PAD, D), jnp.float32),
            pltpu.SemaphoreType.DMA,
            pltpu.SemaphoreType.DMA,
        ],
    )
    eidx = jnp.stack([src_p.reshape(-1, CH), dst_p.reshape(-1, CH)], axis=1)
    return kfn(h, eidx, zeros)


def _linear_body(x_ref, w_ref, b_ref, o_ref):
    o_ref[...] = jnp.dot(x_ref[...], w_ref[...],
                         preferred_element_type=jnp.float32) + b_ref[...]


def _linear(x, w, b):
    n = x.shape[0]
    return pl.pallas_call(
        _linear_body,
        out_shape=jax.ShapeDtypeStruct((n, w.shape[1]), jnp.float32),
    )(x, w, b.reshape(1, -1))


def _bn(h, g, e):
    m = jnp.mean(h, axis=0, keepdims=True)
    v = jnp.mean(jnp.square(h - m), axis=0, keepdims=True)
    return (h - m) * (g * lax.rsqrt(v + BN_EPS)) + e


def _mlp_body(h_ref, p0_ref, p1_ref, w1_ref, b1_ref, g1_ref, e1_ref,
              w2_ref, b2_ref, g2_ref, e2_ref, o_ref):
    z = h_ref[...] + p0_ref[...] + p1_ref[...]
    h1 = jnp.dot(z, w1_ref[...], preferred_element_type=jnp.float32) + b1_ref[...]
    h1 = jnp.maximum(_bn(h1, g1_ref[...], e1_ref[...]), 0.0)
    h2 = jnp.dot(h1, w2_ref[...], preferred_element_type=jnp.float32) + b2_ref[...]
    o_ref[...] = jnp.maximum(_bn(h2, g2_ref[...], e2_ref[...]), 0.0)


def _mlp(h, p0, p1, p):
    n = h.shape[0]
    r = lambda a: a.reshape(1, -1)
    return pl.pallas_call(
        _mlp_body,
        out_shape=jax.ShapeDtypeStruct((n, p['W2'].shape[1]), jnp.float32),
    )(h, p0, p1, p['W1'], r(p['b1']), r(p['g1']), r(p['be1']),
      p['W2'], r(p['b2']), r(p['g2']), r(p['be2']))


def _final_body(h_ref, m_ref, w_ref, b_ref, o_ref):
    z = h_ref[...] * m_ref[...]
    o_ref[...] = jnp.dot(z, w_ref[...],
                         preferred_element_type=jnp.float32) + b_ref[...]


def _final(h, maskf, w, b):
    n = h.shape[0]
    return pl.pallas_call(
        _final_body,
        out_shape=jax.ShapeDtypeStruct((n, w.shape[1]), jnp.float32),
    )(h, maskf, w, b.reshape(1, -1))


def kernel(x, edge_index, mask, params):
    n = x.shape[0]
    e = edge_index.shape[1]
    src = edge_index[0].astype(jnp.int32)
    dst = edge_index[1].astype(jnp.int32)
    epw = NW * CH * 2
    e_pad = ((e + epw - 1) // epw) * epw
    pad = e_pad - e
    if pad:
        src = jnp.concatenate([src, jnp.zeros((pad,), jnp.int32)])
        dst = jnp.concatenate([dst, jnp.full((pad,), N_PAD - 8, jnp.int32)])
    zeros = jnp.zeros((RPT, D), jnp.float32)

    h = _linear(x, params['init_W'], params['init_b'])
    for p in params['convs']:
        parts = _segment_partials(h, src, dst, zeros)
        h = _mlp(h, parts[0:n], parts[N_PAD:N_PAD + n], p)

    maskf = mask.astype(jnp.float32)[:, None]
    wp = jnp.pad(params['lin_W'], ((0, 0), (0, D - N_CLASSES)))
    bp = jnp.pad(params['lin_b'], (0, D - N_CLASSES))
    out = _final(h, maskf, wp, bp)
    return out[:, :N_CLASSES]


# speedup vs baseline: 5.1212x; 1.0760x over previous
import functools

import jax
import jax.numpy as jnp
from jax import lax
from jax.experimental import pallas as pl
from jax.experimental.pallas import tpu as pltpu
from jax.experimental.pallas import tpu_sc as plsc

N_NODES = 10000
D = 128
N_CLASSES = 10
BN_EPS = 1e-5

NC = 2
NS = 16
NW = NC * NS

N_PAD = 10240
RPT = N_PAD // NS
CH = 120


def _seg_body(h_hbm, eidx_hbm, zeros_hbm, out_hbm,
              ia, ib, ra, rb, acc, ga, gb, *, gc0, gc1):
    idx = [ia, ib]
    rows = [ra, rb]
    sems = [ga, gb]
    cid = lax.axis_index("c")
    sid = lax.axis_index("s")
    pltpu.sync_copy(zeros_hbm, acc.at[pl.ds(sid * RPT, RPT)])
    plsc.subcore_barrier()
    gch = jnp.where(cid == 0, gc0, gc1)
    ibase = jnp.where(cid == 0, sid * gc0, NS * gc0 + sid * gc1)

    pltpu.sync_copy(eidx_hbm.at[ibase], ia)
    pltpu.async_copy(h_hbm.at[ia.at[0]], ra, ga)

    def round_body(t, carry):
        for b in range(2):
            g = t * 2 + b
            cur, nxt = b, 1 - b

            @pl.when(g + 1 < gch)
            def _():
                pltpu.sync_copy(eidx_hbm.at[ibase + g + 1], idx[nxt])
                pltpu.async_copy(h_hbm.at[idx[nxt].at[0]], rows[nxt],
                                 sems[nxt])

            pltpu.make_async_copy(h_hbm.at[idx[cur].at[0]], rows[cur],
                                  sems[cur]).wait()
            pltpu.sync_copy(rows[cur], acc.at[idx[cur].at[1]], add=True)
        return carry

    lax.fori_loop(0, gch // 2, round_body, 0)
    plsc.subcore_barrier()
    pltpu.sync_copy(acc.at[pl.ds(sid * RPT, RPT)],
                    out_hbm.at[pl.ds(cid * N_PAD + sid * RPT, RPT)])


CORE0_FRAC = 0.62


def _segment_partials(h, src_p, dst_p, zeros):
    tot = src_p.shape[0] // (NS * CH)
    gc0 = int(round(tot * CORE0_FRAC / 2)) * 2
    gc1 = tot - gc0
    mesh = plsc.VectorSubcoreMesh(core_axis_name="c", subcore_axis_name="s")
    kfn = pl.kernel(
        functools.partial(_seg_body, gc0=gc0, gc1=gc1),
        out_type=jax.ShapeDtypeStruct((NC * N_PAD, D), jnp.float32),
        mesh=mesh,
        scratch_types=[
            pltpu.VMEM((2, CH), jnp.int32),
            pltpu.VMEM((2, CH), jnp.int32),
            pltpu.VMEM((CH, D), jnp.float32),
            pltpu.VMEM((CH, D), jnp.float32),
            pltpu.VMEM_SHARED((N_PAD, D), jnp.float32),
            pltpu.SemaphoreType.DMA,
            pltpu.SemaphoreType.DMA,
        ],
    )
    eidx = jnp.stack([src_p.reshape(-1, CH), dst_p.reshape(-1, CH)], axis=1)
    return kfn(h, eidx, zeros)


def _linear_body(x_ref, w_ref, b_ref, o_ref):
    o_ref[...] = jnp.dot(x_ref[...], w_ref[...],
                         preferred_element_type=jnp.float32) + b_ref[...]


def _linear(x, w, b):
    n = x.shape[0]
    return pl.pallas_call(
        _linear_body,
        out_shape=jax.ShapeDtypeStruct((n, w.shape[1]), jnp.float32),
    )(x, w, b.reshape(1, -1))


def _bn(h, g, e):
    m = jnp.mean(h, axis=0, keepdims=True)
    v = jnp.mean(jnp.square(h - m), axis=0, keepdims=True)
    return (h - m) * (g * lax.rsqrt(v + BN_EPS)) + e


def _mlp_body(h_ref, p0_ref, p1_ref, w1_ref, b1_ref, g1_ref, e1_ref,
              w2_ref, b2_ref, g2_ref, e2_ref, o_ref):
    z = h_ref[...] + p0_ref[...] + p1_ref[...]
    h1 = jnp.dot(z, w1_ref[...], preferred_element_type=jnp.float32) + b1_ref[...]
    h1 = jnp.maximum(_bn(h1, g1_ref[...], e1_ref[...]), 0.0)
    h2 = jnp.dot(h1, w2_ref[...], preferred_element_type=jnp.float32) + b2_ref[...]
    o_ref[...] = jnp.maximum(_bn(h2, g2_ref[...], e2_ref[...]), 0.0)


def _mlp(h, p0, p1, p):
    n = h.shape[0]
    r = lambda a: a.reshape(1, -1)
    return pl.pallas_call(
        _mlp_body,
        out_shape=jax.ShapeDtypeStruct((n, p['W2'].shape[1]), jnp.float32),
    )(h, p0, p1, p['W1'], r(p['b1']), r(p['g1']), r(p['be1']),
      p['W2'], r(p['b2']), r(p['g2']), r(p['be2']))


def _final_body(h_ref, m_ref, w_ref, b_ref, o_ref):
    z = h_ref[...] * m_ref[...]
    o_ref[...] = jnp.dot(z, w_ref[...],
                         preferred_element_type=jnp.float32) + b_ref[...]


def _final(h, maskf, w, b):
    n = h.shape[0]
    return pl.pallas_call(
        _final_body,
        out_shape=jax.ShapeDtypeStruct((n, w.shape[1]), jnp.float32),
    )(h, maskf, w, b.reshape(1, -1))


def kernel(x, edge_index, mask, params):
    n = x.shape[0]
    e = edge_index.shape[1]
    src = edge_index[0].astype(jnp.int32)
    dst = edge_index[1].astype(jnp.int32)
    epw = NW * CH * 2
    e_pad = ((e + epw - 1) // epw) * epw
    pad = e_pad - e
    if pad:
        src = jnp.concatenate([src, jnp.zeros((pad,), jnp.int32)])
        dst = jnp.concatenate([dst, jnp.full((pad,), N_PAD - 8, jnp.int32)])
    zeros = jnp.zeros((RPT, D), jnp.float32)

    h = _linear(x, params['init_W'], params['init_b'])
    for p in params['convs']:
        parts = _segment_partials(h, src, dst, zeros)
        h = _mlp(h, parts[0:n], parts[N_PAD:N_PAD + n], p)

    maskf = mask.astype(jnp.float32)[:, None]
    wp = jnp.pad(params['lin_W'], ((0, 0), (0, D - N_CLASSES)))
    bp = jnp.pad(params['lin_b'], (0, D - N_CLASSES))
    out = _final(h, maskf, wp, bp)
    return out[:, :N_CLASSES]
